# baseline (speedup 1.0000x reference)
import functools

import jax
import jax.numpy as jnp
from jax import lax
from jax.experimental import pallas as pl
from jax.experimental.pallas import tpu as pltpu
from jax.experimental.pallas import tpu_sc as plsc

N = 8192
GEMV_BM = 256
ROWS = 64
COLS = 128
NEG_INF = float("-inf")


NSPLIT = 2


def _fused_kernel(*refs):
    x_refs = refs[:NSPLIT]
    w_refs = refs[NSPLIT:2 * NSPLIT]
    b_ref, c_ref, ys_ref, m_ref, h_scr = refs[2 * NSPLIT:]
    acc = b_ref[...]
    for xr, wr in zip(x_refs, w_refs):
        acc = acc + jnp.dot(wr[...], xr[...],
                            preferred_element_type=jnp.float32)
    i = pl.program_id(0)
    rows_per_blk = GEMV_BM // COLS
    h_scr[pl.ds(i * rows_per_blk, rows_per_blk), :] = acc.reshape(
        rows_per_blk, COLS)

    @pl.when(i == (N // GEMV_BM) - 1)
    def _():
        _bcsoftmax(h_scr[...], c_ref[...], ys_ref, m_ref)


def _iotas():
    row = lax.broadcasted_iota(jnp.int32, (ROWS, COLS), 0)
    lane = lax.broadcasted_iota(jnp.int32, (ROWS, COLS), 1)
    return row, lane


def _make_partner(row, lane, j):
    if j < ROWS:
        bit = (row & j) != 0
        return lambda a: jnp.where(
            bit, jnp.roll(a, j, axis=0), jnp.roll(a, -j, axis=0))
    r = j // ROWS
    bit = (lane & r) != 0
    return lambda a: jnp.where(
        bit, jnp.roll(a, r, axis=1), jnp.roll(a, -r, axis=1))


def _stage_masks(row, lane, j, k):
    low = (row & j) == 0 if j < ROWS else (lane & (j // ROWS)) == 0
    asc = (row & k) == 0 if k < ROWS else (lane & (k // ROWS)) == 0
    return low == asc


def _bitonic_sort(key, idx, payloads):
    row, lane = _iotas()
    n = ROWS * COLS
    k = 2
    while k <= n:
        j = k // 2
        while j >= 1:
            partner = _make_partner(row, lane, j)
            kp = partner(key)
            ip = partner(idx)
            lt = jnp.logical_or(
                key < kp, jnp.logical_and(key == kp, idx < ip))
            take_self = _stage_masks(row, lane, j, k) == lt
            key = jnp.where(take_self, key, kp)
            idx = jnp.where(take_self, idx, ip)
            payloads = [jnp.where(take_self, a, partner(a))
                        for a in payloads]
            j //= 2
        k *= 2
    return key, idx, payloads


def _bitonic_unsort(idx, y):
    row, lane = _iotas()
    n = ROWS * COLS
    k = 2
    while k <= n:
        j = k // 2
        while j >= 1:
            partner = _make_partner(row, lane, j)
            ip = partner(idx)
            take_self = _stage_masks(row, lane, j, k) == (idx < ip)
            idx = jnp.where(take_self, idx, ip)
            y = jnp.where(take_self, y, partner(y))
            j //= 2
        k *= 2
    return y


def _flat_cumsum(a):
    row = lax.broadcasted_iota(jnp.int32, (ROWS, COLS), 0)
    x = a
    d = 1
    while d < ROWS:
        x = x + jnp.where(row >= d, jnp.roll(x, d, axis=0), 0.0)
        d *= 2
    lane1 = lax.broadcasted_iota(jnp.int32, (1, COLS), 1)
    tot = x[ROWS - 1:ROWS, :]
    y = tot
    d = 1
    while d < COLS:
        y = y + jnp.where(lane1 >= d, jnp.roll(y, d, axis=1), 0.0)
        d *= 2
    return x + (y - tot)


def _bcsoftmax(h, c, ys_ref, m_ref):
    logc = jnp.where(c == 0.0, -1.0e38, jnp.log(c))
    key = logc - h
    row, lane = _iotas()
    idx = row * COLS + lane

    ks, idx_s, (bs,) = _bitonic_sort(key, idx, [c])
    lb = jnp.log(bs)
    xs = lb - ks

    mx = jnp.max(xs)
    e = jnp.exp(xs - mx)
    etot = jnp.sum(e)
    s = 1.0 - (_flat_cumsum(bs) - bs)
    r_ge = etot - (_flat_cumsum(e) - e)
    logr = mx + jnp.log(r_ge)
    in_kb = jnp.logical_or(
        bs == 0.0,
        jnp.logical_and(s - bs > 0.0,
                        xs - logr + jnp.log(s) > lb),
    )
    m2 = jnp.max(jnp.where(in_kb, NEG_INF, xs))
    ex = jnp.exp(xs - m2)
    s2 = 1.0 - jnp.sum(jnp.where(in_kb, bs, 0.0))
    r = jnp.sum(jnp.where(in_kb, 0.0, ex))
    ys = jnp.where(in_kb, bs, s2 * ex / r)

    ys_ref[...] = ys
    m_ref[...] = idx_s


_SC_CHUNK = 128
_SC_WORKERS = 32


@functools.partial(
    pl.kernel,
    mesh=plsc.VectorSubcoreMesh(core_axis_name="c", subcore_axis_name="s"),
    out_type=jax.ShapeDtypeStruct((N,), jnp.float32),
    scratch_types=[
        pltpu.VMEM((_SC_CHUNK,), jnp.int32),
        pltpu.VMEM((_SC_CHUNK,), jnp.float32),
        pltpu.SemaphoreType.DMA,
    ],
)
def _sc_scatter(m_hbm, ys_hbm, out_hbm, idx_v, val_v, sem):
    wid = lax.axis_index("s") * 2 + lax.axis_index("c")
    per_w = N // _SC_WORKERS
    for sub in range(per_w // _SC_CHUNK):
        off = wid * per_w + sub * _SC_CHUNK
        pltpu.sync_copy(m_hbm.at[pl.ds(off, _SC_CHUNK)], idx_v)
        pltpu.sync_copy(ys_hbm.at[pl.ds(off, _SC_CHUNK)], val_v)
        pltpu.async_copy(val_v, out_hbm.at[idx_v], sem).wait()


@jax.jit
def kernel(x, c, W, b):
    x2 = x.reshape(N, 1)
    b2 = b.reshape(N, 1)
    ys2, m2 = pl.pallas_call(
        _fused_kernel,
        grid=(N // GEMV_BM,),
        in_specs=(
            [pl.BlockSpec((N // NSPLIT, 1), (lambda s: lambda i: (s, 0))(s))
             for s in range(NSPLIT)]
            + [pl.BlockSpec((GEMV_BM, N // NSPLIT),
                            (lambda s: lambda i: (i, s))(s))
               for s in range(NSPLIT)]
            + [pl.BlockSpec((GEMV_BM, 1), lambda i: (i, 0)),
               pl.BlockSpec((ROWS, COLS), lambda i: (0, 0))]
        ),
        out_specs=[pl.BlockSpec((ROWS, COLS), lambda i: (0, 0)),
                   pl.BlockSpec((ROWS, COLS), lambda i: (0, 0))],
        out_shape=[jax.ShapeDtypeStruct((ROWS, COLS), jnp.float32),
                   jax.ShapeDtypeStruct((ROWS, COLS), jnp.int32)],
        scratch_shapes=[pltpu.VMEM((ROWS, COLS), jnp.float32)],
    )(*([x2] * NSPLIT + [W] * NSPLIT + [b2, c.reshape(ROWS, COLS)]))
    return _sc_scatter(m2.reshape(N), ys2.reshape(N))


# speedup vs baseline: 1.4324x; 1.4324x over previous
import jax
import jax.numpy as jnp
from jax import lax
from jax.experimental import pallas as pl
from jax.experimental.pallas import tpu as pltpu

N = 8192
GEMV_BM = 256
ROWS = 64
COLS = 128
NEG_INF = float("-inf")


NSPLIT = 2


def _fused_kernel(*refs):
    x_refs = refs[:NSPLIT]
    w_refs = refs[NSPLIT:2 * NSPLIT]
    b_ref, c_ref, o_ref, h_scr = refs[2 * NSPLIT:]
    acc = b_ref[...]
    for xr, wr in zip(x_refs, w_refs):
        acc = acc + jnp.dot(wr[...], xr[...],
                            preferred_element_type=jnp.float32)
    i = pl.program_id(0)
    rows_per_blk = GEMV_BM // COLS
    h_scr[pl.ds(i * rows_per_blk, rows_per_blk), :] = acc.reshape(
        rows_per_blk, COLS)

    @pl.when(i == (N // GEMV_BM) - 1)
    def _():
        _bcsoftmax(h_scr[...], c_ref[...], o_ref)


def _iotas():
    row = lax.broadcasted_iota(jnp.int32, (ROWS, COLS), 0)
    lane = lax.broadcasted_iota(jnp.int32, (ROWS, COLS), 1)
    return row, lane


def _make_partner(row, lane, j):
    if j < ROWS:
        bit = (row & j) != 0
        return lambda a: jnp.where(
            bit, jnp.roll(a, j, axis=0), jnp.roll(a, -j, axis=0))
    r = j // ROWS
    bit = (lane & r) != 0
    return lambda a: jnp.where(
        bit, jnp.roll(a, r, axis=1), jnp.roll(a, -r, axis=1))


def _stage_masks(row, lane, j, k):
    low = (row & j) == 0 if j < ROWS else (lane & (j // ROWS)) == 0
    asc = (row & k) == 0 if k < ROWS else (lane & (k // ROWS)) == 0
    return low == asc


def _bitonic_sort(key, idx, payloads):
    row, lane = _iotas()
    n = ROWS * COLS
    k = 2
    while k <= n:
        j = k // 2
        while j >= 1:
            partner = _make_partner(row, lane, j)
            kp = partner(key)
            ip = partner(idx)
            lt = jnp.logical_or(
                key < kp, jnp.logical_and(key == kp, idx < ip))
            take_self = _stage_masks(row, lane, j, k) == lt
            key = jnp.where(take_self, key, kp)
            idx = jnp.where(take_self, idx, ip)
            payloads = [jnp.where(take_self, a, partner(a))
                        for a in payloads]
            j //= 2
        k *= 2
    return key, idx, payloads


def _bitonic_unsort(idx, y):
    row, lane = _iotas()
    n = ROWS * COLS
    k = 2
    while k <= n:
        j = k // 2
        while j >= 1:
            partner = _make_partner(row, lane, j)
            ip = partner(idx)
            take_self = _stage_masks(row, lane, j, k) == (idx < ip)
            idx = jnp.where(take_self, idx, ip)
            y = jnp.where(take_self, y, partner(y))
            j //= 2
        k *= 2
    return y


def _flat_cumsum(a):
    row = lax.broadcasted_iota(jnp.int32, (ROWS, COLS), 0)
    x = a
    d = 1
    while d < ROWS:
        x = x + jnp.where(row >= d, jnp.roll(x, d, axis=0), 0.0)
        d *= 2
    lane1 = lax.broadcasted_iota(jnp.int32, (1, COLS), 1)
    tot = x[ROWS - 1:ROWS, :]
    y = tot
    d = 1
    while d < COLS:
        y = y + jnp.where(lane1 >= d, jnp.roll(y, d, axis=1), 0.0)
        d *= 2
    return x + (y - tot)


def _bcsoftmax(h, c, o_ref):
    logc = jnp.where(c == 0.0, -1.0e38, jnp.log(c))
    key = logc - h
    row, lane = _iotas()
    idx = row * COLS + lane

    ks, idx_s, (bs,) = _bitonic_sort(key, idx, [c])
    lb = jnp.log(bs)
    xs = lb - ks

    mx = jnp.max(xs)
    e = jnp.exp(xs - mx)
    etot = jnp.sum(e)
    s = 1.0 - (_flat_cumsum(bs) - bs)
    r_ge = etot - (_flat_cumsum(e) - e)
    logr = mx + jnp.log(r_ge)
    in_kb = jnp.logical_or(
        bs == 0.0,
        jnp.logical_and(s - bs > 0.0,
                        xs - logr + jnp.log(s) > lb),
    )
    m2 = jnp.max(jnp.where(in_kb, NEG_INF, xs))
    ex = jnp.exp(xs - m2)
    s2 = 1.0 - jnp.sum(jnp.where(in_kb, bs, 0.0))
    r = jnp.sum(jnp.where(in_kb, 0.0, ex))
    ys = jnp.where(in_kb, bs, s2 * ex / r)

    m = ((idx_s & (COLS - 1)) << 6) | (idx_s >> 7)
    o_ref[...] = _bitonic_unsort(m, ys)


@jax.jit
def kernel(x, c, W, b):
    x2 = x.reshape(N, 1)
    b2 = b.reshape(N, 1)
    y2 = pl.pallas_call(
        _fused_kernel,
        grid=(N // GEMV_BM,),
        in_specs=(
            [pl.BlockSpec((N // NSPLIT, 1), (lambda s: lambda i: (s, 0))(s))
             for s in range(NSPLIT)]
            + [pl.BlockSpec((GEMV_BM, N // NSPLIT),
                            (lambda s: lambda i: (i, s))(s))
               for s in range(NSPLIT)]
            + [pl.BlockSpec((GEMV_BM, 1), lambda i: (i, 0)),
               pl.BlockSpec((ROWS, COLS), lambda i: (0, 0))]
        ),
        out_specs=pl.BlockSpec((ROWS, COLS), lambda i: (0, 0)),
        out_shape=jax.ShapeDtypeStruct((ROWS, COLS), jnp.float32),
        scratch_shapes=[pltpu.VMEM((ROWS, COLS), jnp.float32)],
    )(*([x2] * NSPLIT + [W] * NSPLIT + [b2, c.reshape(ROWS, COLS)]))
    return y2.reshape(N)
